# baseline (speedup 1.0000x reference)
import functools

import jax
import jax.numpy as jnp
from jax import lax
from jax.experimental import pallas as pl
from jax.experimental.pallas import tpu as pltpu
from jax.experimental.pallas import tpu_sc as plsc

_VOCAB = 1_000_000
_D = 64
_BATCH = 4096
_SEQ = 200
_NW = 32
_RPW = _BATCH // _NW
_NBUF = 2
_NSTEP = _RPW // _NBUF

_mesh = plsc.VectorSubcoreMesh(core_axis_name="c", subcore_axis_name="s")


@functools.partial(
    pl.kernel,
    out_type=jax.ShapeDtypeStruct((_BATCH, _SEQ, _D), jnp.float32),
    mesh=_mesh,
    scratch_types=[
        [pltpu.VMEM((_SEQ,), jnp.int32) for _ in range(_NBUF)],
        [pltpu.VMEM((_SEQ, _D), jnp.float32) for _ in range(_NBUF)],
        [pltpu.SemaphoreType.DMA for _ in range(_NBUF)],
        [pltpu.SemaphoreType.DMA for _ in range(_NBUF)],
        [pltpu.SemaphoreType.DMA for _ in range(_NBUF)],
    ],
    compiler_params=pltpu.CompilerParams(use_tc_tiling_on_sc=False),
)
def _gather_kernel(idx_hbm, table_hbm, out_hbm, idx_v, rows_v, sem_i, sem_g, sem_s):
    wid = lax.axis_index("s") * 2 + lax.axis_index("c")
    base = wid * _RPW

    def idx_copy(b, row):
        return pltpu.make_async_copy(idx_hbm.at[row], idx_v[b], sem_i[b])

    def gather_copy(b):
        return pltpu.make_async_copy(table_hbm.at[idx_v[b]], rows_v[b], sem_g[b])

    def store_copy(b, row):
        return pltpu.make_async_copy(rows_v[b], out_hbm.at[row], sem_s[b])

    for b in range(_NBUF):
        idx_copy(b, base + b).start()
    for b in range(_NBUF):
        idx_copy(b, base + b).wait()
        gather_copy(b).start()

    def body(g, carry):
        for b in range(_NBUF):
            row_prev = base + (g - 1) * _NBUF + b
            row_new = base + g * _NBUF + b
            gather_copy(b).wait()
            store_copy(b, row_prev).start()
            idx_copy(b, row_new).start()
            store_copy(b, row_prev).wait()
            idx_copy(b, row_new).wait()
            gather_copy(b).start()
        return carry

    lax.fori_loop(1, _NSTEP, body, 0)

    for b in range(_NBUF):
        row = base + (_NSTEP - 1) * _NBUF + b
        gather_copy(b).wait()
        store_copy(b, row).start()
    for b in range(_NBUF):
        row = base + (_NSTEP - 1) * _NBUF + b
        store_copy(b, row).wait()


def kernel(x, embedding_table, training, mask):
    return _gather_kernel(x.astype(jnp.int32), embedding_table)
